# baseline (speedup 1.0000x reference)
import jax
import jax.numpy as jnp
import numpy as np
from jax.experimental import pallas as pl
from jax.experimental.pallas import tpu as pltpu

C = 640
KD = 160
R = 4
VR = 8
B = 4
H = 64
W = 64
HW = H * W
GROUPS = 32
GC = C // GROUPS
PAD = 72
SROWS = PAD + HW + PAD

_SQRT2 = np.sqrt(2.0).astype(np.float32)


def _gelu(x):
    return 0.5 * x * (1.0 + jax.lax.erf(x / _SQRT2))


def _mm(a, b):
    return jax.lax.dot_general(a, b, (((1,), (0,)), ((), ())),
                               preferred_element_type=jnp.float32)


def _mm_bf(a, b):
    return jax.lax.dot_general(a.astype(jnp.bfloat16), b,
                               (((1,), (0,)), ((), ())),
                               preferred_element_type=jnp.float32)


def _mm_lT(a, b):
    return jax.lax.dot_general(a, b, (((0,), (0,)), ((), ())),
                               preferred_element_type=jnp.float32)


def _mm_rT(a, b):
    return jax.lax.dot_general(a, b, (((1,), (1,)), ((), ())),
                               preferred_element_type=jnp.float32)


CH = 1024


def _zero_scratch_pads(scr_ref):
    cb = scr_ref.shape[1]
    z = jnp.zeros((PAD, cb), scr_ref.dtype)
    scr_ref[pl.ds(0, PAD), :] = z
    scr_ref[pl.ds(PAD + HW, PAD), :] = z


def _dwconv_chunk(scr_ref, s0, taps):
    row = jax.lax.broadcasted_iota(jnp.int32, (CH, 1), 0) % W
    mask_l = row != 0
    mask_r = row != W - 1

    def ld(d):
        return scr_ref[pl.ds(PAD + s0 + d, CH), :].astype(jnp.float32)

    acc_l = None
    acc_c = None
    acc_r = None
    for dh in (-1, 0, 1):
        tl = taps[3 * (dh + 1) + 0][None, :]
        tc = taps[3 * (dh + 1) + 1][None, :]
        tr = taps[3 * (dh + 1) + 2][None, :]
        vl = ld(dh * W - 1) * tl
        vc = ld(dh * W) * tc
        vr = ld(dh * W + 1) * tr
        acc_l = vl if acc_l is None else acc_l + vl
        acc_c = vc if acc_c is None else acc_c + vc
        acc_r = vr if acc_r is None else acc_r + vr
    return acc_c + jnp.where(mask_l, acc_l, 0.0) + jnp.where(mask_r, acc_r, 0.0)


def _qproj_body(cc_ref, wq_ref, bq_ref, q_ref):
    x = cc_ref[0].astype(jnp.bfloat16)
    p = _mm(wq_ref[...].astype(jnp.bfloat16), x) + bq_ref[...]
    g = _gelu(p)
    q_ref[0] = jnp.sum(g, axis=1, keepdims=True).T


def _route_body(q_ref, keys_ref, wsel_ref):
    q = q_ref[:, 0, :] * (1.0 / HW)
    qn = q / jnp.maximum(jnp.sqrt(jnp.sum(q * q, axis=1, keepdims=True)), 1e-12)
    kk = keys_ref[...]
    kn = kk / jnp.maximum(jnp.sqrt(jnp.sum(kk * kk, axis=1, keepdims=True)), 1e-12)
    sims = jax.lax.dot_general(qn, kn, (((1,), (1,)), ((), ())),
                               preferred_element_type=jnp.float32)
    r_iota = jax.lax.broadcasted_iota(jnp.int32, (B, R), 1)
    m1 = jnp.max(sims, axis=1, keepdims=True)
    i1 = jnp.min(jnp.where(sims == m1, r_iota, R), axis=1, keepdims=True)
    sel1 = (r_iota == i1)
    sims2 = jnp.where(sel1, -1e30, sims)
    m2 = jnp.max(sims2, axis=1, keepdims=True)
    i2 = jnp.min(jnp.where(sims2 == m2, r_iota, R), axis=1, keepdims=True)
    sel2 = (r_iota == i2)
    e2 = jnp.exp((m2 - m1) * 10.0)
    a1 = 1.0 / (1.0 + e2)
    a2 = e2 / (1.0 + e2)
    wsel_ref[...] = jnp.where(sel1, a1, 0.0) + jnp.where(sel2, a2, 0.0)


def _decode_body(vals_ref, w1_ref, b1_ref, khw_ref, wsel_ref, w9_ref, b2_ref,
                 out_ref, s0_ref, s1_ref, s2_ref, s3_ref):
    i = pl.program_id(0)
    scrs = [s0_ref, s1_ref, s2_ref, s3_ref]

    @pl.when(i == 0)
    def _zero_pads():
        for sr in scrs:
            _zero_scratch_pads(sr)

    t = _mm(vals_ref[...], w1_ref[...]) + b1_ref[...]
    tb = t.astype(jnp.bfloat16)
    trs = [tb[r * 64:(r + 1) * 64, :] for r in range(R)]
    for s in range(HW // CH):
        ku = khw_ref[pl.ds(s * CH, CH), :]
        gs = [_gelu(_mm(ku, trs[r])) for r in range(R)]
        for b in range(B):
            wb = None
            for r in range(R):
                term = gs[r] * wsel_ref[b, r]
                wb = term if wb is None else wb + term
            scrs[b][pl.ds(PAD + s * CH, CH), :] = wb
    taps = w9_ref[...]
    b2 = b2_ref[...]
    for b in range(B):
        for s in range(HW // CH):
            y = _dwconv_chunk(scrs[b], s * CH, taps) + b2
            out_ref[b, pl.ds(s * CH, CH), :] = y.astype(jnp.bfloat16)


def _fusion1_body(cc_ref, wv_ref, w1a_ref, w1b_ref, b1_ref, fp_ref, st_ref):
    x = cc_ref[0].astype(jnp.bfloat16)
    wv = wv_ref[0]
    f = _mm_lT(x, w1a_ref[...]) + _mm(wv, w1b_ref[...]) + b1_ref[...]
    fp_ref[0] = f.astype(jnp.bfloat16)
    s0 = jnp.sum(f, axis=0, keepdims=True)
    s1 = jnp.sum(f * f, axis=0, keepdims=True)
    upd = jnp.concatenate([s0, s1], axis=1)
    s_idx = pl.program_id(1)

    @pl.when(s_idx == 0)
    def _init():
        st_ref[0] = upd

    @pl.when(s_idx != 0)
    def _acc():
        st_ref[0] = st_ref[0] + upd


def _fusion2_body(fp_ref, st_ref, gsm_ref, gamma_ref, beta_ref, w9_ref, b2_ref,
                  out_ref, scr_ref):
    b = pl.program_id(0)

    @pl.when(b == 0)
    def _zero_pads():
        _zero_scratch_pads(scr_ref)

    st = st_ref[0]
    gsm = gsm_ref[...]
    gs0 = _mm(st[:, :C], gsm)
    gs1 = _mm(st[:, C:], gsm)
    inv_n = 1.0 / (GC * HW)
    mean_g = gs0 * inv_n
    var_g = gs1 * inv_n - mean_g * mean_g
    rstd_g = jax.lax.rsqrt(var_g + 1e-5)
    mean_c = _mm(mean_g, gsm.T)
    rstd_c = _mm(rstd_g, gsm.T)
    sg = rstd_c * gamma_ref[...]
    off = beta_ref[...] - mean_c * sg
    for s in range(HW // CH):
        f = fp_ref[0, pl.ds(s * CH, CH), :].astype(jnp.float32)
        y = _gelu(f * sg + off)
        scr_ref[pl.ds(PAD + s * CH, CH), :] = y
    taps = w9_ref[...]
    b2 = b2_ref[...]
    for s in range(HW // CH):
        y = _dwconv_chunk(scr_ref, s * CH, taps) + b2
        out_ref[0, pl.ds(s * CH, CH), :] = y.astype(jnp.bfloat16)


def _final_body(fm_ref, cc_ref, wv_ref, w3_ref, b3_ref, wga_ref, wgb_ref,
                bg_ref, out_ref):
    x = cc_ref[0]
    xb = x.astype(jnp.bfloat16)
    fo = _mm_rT(w3_ref[...], fm_ref[0]) + b3_ref[...]
    gp = _mm(wga_ref[...], xb) + _mm_rT(wgb_ref[...], wv_ref[0]) + bg_ref[...]
    gate = jax.nn.sigmoid(gp)
    out_ref[0] = gate * fo + (1.0 - gate) * x


def _full(shape):
    nd = len(shape)
    return pl.BlockSpec(shape, lambda *a: (0,) * nd)


def kernel(current_context, k, Wq, bq, memory_keys, memory_values,
           vd_w1, vd_b1, vd_w2, vd_b2, fn_w1, fn_b1, gn_gamma, gn_beta,
           fn_w2, fn_b2, fn_w3, fn_b3, fg_w, fg_b):
    del k
    f32 = jnp.float32
    bf16 = jnp.bfloat16
    cc = current_context.reshape(B, C, HW)

    U = jax.image.resize(jnp.eye(VR, dtype=f32), (H, VR), method='bilinear')
    khw = jnp.kron(U, U).astype(bf16)
    vals2t = memory_values[0].transpose(0, 2, 3, 1).reshape(R * VR * VR, C)
    keys2 = memory_keys[0, :, :, 0, 0]
    bq2 = bq.reshape(KD, 1)
    w1T = vd_w1.T
    vd_b1r = vd_b1.reshape(1, C)
    vd_w9 = vd_w2.reshape(C, 9).T
    vd_b2r = vd_b2.reshape(1, C)
    w1aT = fn_w1[:, :C].T.astype(bf16)
    w1bT = fn_w1[:, C:].T.astype(bf16)
    fn_b1r = fn_b1.reshape(1, C)
    gsm = jnp.kron(jnp.eye(GROUPS, dtype=f32), jnp.ones((GC, 1), f32))
    gammar = gn_gamma.reshape(1, C)
    betar = gn_beta.reshape(1, C)
    fn_w9 = fn_w2.reshape(C, 9).T
    fn_b2r = fn_b2.reshape(1, C)
    w3b = fn_w3.astype(bf16)
    fn_b3c = fn_b3.reshape(C, 1)
    wgab = fg_w[:, :C].astype(bf16)
    wgbb = fg_w[:, C:].astype(bf16)
    fg_bc = fg_b.reshape(C, 1)

    qsum = pl.pallas_call(
        _qproj_body,
        grid=(B,),
        in_specs=[
            pl.BlockSpec((1, C, HW), lambda b: (b, 0, 0)),
            pl.BlockSpec((KD, C), lambda b: (0, 0)),
            pl.BlockSpec((KD, 1), lambda b: (0, 0)),
        ],
        out_specs=pl.BlockSpec((1, 1, KD), lambda b: (b, 0, 0)),
        out_shape=jax.ShapeDtypeStruct((B, 1, KD), f32),
    )(cc, Wq, bq2)

    wsel = pl.pallas_call(
        _route_body,
        in_specs=[_full((B, 1, KD)), _full((R, KD))],
        out_specs=_full((B, R)),
        out_shape=jax.ShapeDtypeStruct((B, R), f32),
    )(qsum, keys2)

    CB = 128
    NCB = C // CB
    weighted = pl.pallas_call(
        _decode_body,
        grid=(NCB,),
        in_specs=[
            pl.BlockSpec((R * VR * VR, C), lambda i: (0, 0)),
            pl.BlockSpec((C, CB), lambda i: (0, i)),
            pl.BlockSpec((1, CB), lambda i: (0, i)),
            pl.BlockSpec((HW, VR * VR), lambda i: (0, 0)),
            pl.BlockSpec(memory_space=pltpu.SMEM),
            pl.BlockSpec((9, CB), lambda i: (0, i)),
            pl.BlockSpec((1, CB), lambda i: (0, i)),
        ],
        out_specs=pl.BlockSpec((B, HW, CB), lambda i: (0, 0, i)),
        out_shape=jax.ShapeDtypeStruct((B, HW, C), bf16),
        scratch_shapes=[pltpu.VMEM((SROWS, CB), f32) for _ in range(B)],
    )(vals2t, w1T, vd_b1r, khw, wsel, vd_w9, vd_b2r)

    SC1 = 2048
    NS1 = HW // SC1
    fusion_pre, stats = pl.pallas_call(
        _fusion1_body,
        grid=(B, NS1),
        in_specs=[
            pl.BlockSpec((1, C, SC1), lambda b, s: (b, 0, s)),
            pl.BlockSpec((1, SC1, C), lambda b, s: (b, s, 0)),
            pl.BlockSpec((C, C), lambda b, s: (0, 0)),
            pl.BlockSpec((C, C), lambda b, s: (0, 0)),
            pl.BlockSpec((1, C), lambda b, s: (0, 0)),
        ],
        out_specs=[
            pl.BlockSpec((1, SC1, C), lambda b, s: (b, s, 0)),
            pl.BlockSpec((1, 1, 2 * C), lambda b, s: (b, 0, 0)),
        ],
        out_shape=[
            jax.ShapeDtypeStruct((B, HW, C), bf16),
            jax.ShapeDtypeStruct((B, 1, 2 * C), f32),
        ],
    )(cc, weighted, w1aT, w1bT, fn_b1r)

    fusion_mid = pl.pallas_call(
        _fusion2_body,
        grid=(B,),
        in_specs=[
            pl.BlockSpec((1, HW, C), lambda b: (b, 0, 0)),
            pl.BlockSpec((1, 1, 2 * C), lambda b: (b, 0, 0)),
            pl.BlockSpec((C, GROUPS), lambda b: (0, 0)),
            pl.BlockSpec((1, C), lambda b: (0, 0)),
            pl.BlockSpec((1, C), lambda b: (0, 0)),
            pl.BlockSpec((9, C), lambda b: (0, 0)),
            pl.BlockSpec((1, C), lambda b: (0, 0)),
        ],
        out_specs=pl.BlockSpec((1, HW, C), lambda b: (b, 0, 0)),
        out_shape=jax.ShapeDtypeStruct((B, HW, C), bf16),
        scratch_shapes=[pltpu.VMEM((SROWS, C), f32)],
    )(fusion_pre, stats, gsm, gammar, betar, fn_w9, fn_b2r)

    SC2 = 1024
    NS2 = HW // SC2
    out = pl.pallas_call(
        _final_body,
        grid=(B, NS2),
        in_specs=[
            pl.BlockSpec((1, SC2, C), lambda b, s: (b, s, 0)),
            pl.BlockSpec((1, C, SC2), lambda b, s: (b, 0, s)),
            pl.BlockSpec((1, SC2, C), lambda b, s: (b, s, 0)),
            pl.BlockSpec((C, C), lambda b, s: (0, 0)),
            pl.BlockSpec((C, 1), lambda b, s: (0, 0)),
            pl.BlockSpec((C, C), lambda b, s: (0, 0)),
            pl.BlockSpec((C, C), lambda b, s: (0, 0)),
            pl.BlockSpec((C, 1), lambda b, s: (0, 0)),
        ],
        out_specs=pl.BlockSpec((1, C, SC2), lambda b, s: (b, 0, s)),
        out_shape=jax.ShapeDtypeStruct((B, C, HW), f32),
    )(fusion_mid, cc, weighted, w3b, fn_b3c, wgab, wgbb, fg_bc)

    return out.reshape(B, C, H, W)


# speedup vs baseline: 1.1164x; 1.1164x over previous
import jax
import jax.numpy as jnp
import numpy as np
from jax.experimental import pallas as pl
from jax.experimental.pallas import tpu as pltpu

C = 640
KD = 160
R = 4
VR = 8
B = 4
H = 64
W = 64
HW = H * W
GROUPS = 32
GC = C // GROUPS
PAD = 72
SROWS = PAD + HW + PAD

_SQRT2 = np.sqrt(2.0).astype(np.float32)


def _gelu(x):
    return 0.5 * x * (1.0 + jax.lax.erf(x / _SQRT2))


def _mm(a, b):
    return jax.lax.dot_general(a, b, (((1,), (0,)), ((), ())),
                               preferred_element_type=jnp.float32)


def _mm_bf(a, b):
    return jax.lax.dot_general(a.astype(jnp.bfloat16), b,
                               (((1,), (0,)), ((), ())),
                               preferred_element_type=jnp.float32)


def _mm_lT(a, b):
    return jax.lax.dot_general(a, b, (((0,), (0,)), ((), ())),
                               preferred_element_type=jnp.float32)


def _mm_rT(a, b):
    return jax.lax.dot_general(a, b, (((1,), (1,)), ((), ())),
                               preferred_element_type=jnp.float32)


CH = 1024


def _zero_scratch_pads(scr_ref):
    cb = scr_ref.shape[1]
    z = jnp.zeros((PAD, cb), scr_ref.dtype)
    scr_ref[pl.ds(0, PAD), :] = z
    scr_ref[pl.ds(PAD + HW, PAD), :] = z


def _dwconv_chunk(scr_ref, s0, taps):
    row = jax.lax.broadcasted_iota(jnp.int32, (CH, 1), 0) % W
    mask_l = row != 0
    mask_r = row != W - 1

    def ld(d):
        return scr_ref[pl.ds(PAD + s0 + d, CH), :].astype(jnp.float32)

    acc_l = None
    acc_c = None
    acc_r = None
    for dh in (-1, 0, 1):
        tl = taps[3 * (dh + 1) + 0][None, :]
        tc = taps[3 * (dh + 1) + 1][None, :]
        tr = taps[3 * (dh + 1) + 2][None, :]
        vl = ld(dh * W - 1) * tl
        vc = ld(dh * W) * tc
        vr = ld(dh * W + 1) * tr
        acc_l = vl if acc_l is None else acc_l + vl
        acc_c = vc if acc_c is None else acc_c + vc
        acc_r = vr if acc_r is None else acc_r + vr
    return acc_c + jnp.where(mask_l, acc_l, 0.0) + jnp.where(mask_r, acc_r, 0.0)


def _qproj_body(cc_ref, wq_ref, bq_ref, q_ref):
    x = cc_ref[0].astype(jnp.bfloat16)
    p = _mm(wq_ref[...].astype(jnp.bfloat16), x) + bq_ref[...]
    g = _gelu(p)
    q_ref[0] = jnp.sum(g, axis=1, keepdims=True).T


def _route_body(q_ref, keys_ref, wsel_ref):
    q = q_ref[:, 0, :] * (1.0 / HW)
    qn = q / jnp.maximum(jnp.sqrt(jnp.sum(q * q, axis=1, keepdims=True)), 1e-12)
    kk = keys_ref[...]
    kn = kk / jnp.maximum(jnp.sqrt(jnp.sum(kk * kk, axis=1, keepdims=True)), 1e-12)
    sims = jax.lax.dot_general(qn, kn, (((1,), (1,)), ((), ())),
                               preferred_element_type=jnp.float32)
    r_iota = jax.lax.broadcasted_iota(jnp.int32, (B, R), 1)
    m1 = jnp.max(sims, axis=1, keepdims=True)
    i1 = jnp.min(jnp.where(sims == m1, r_iota, R), axis=1, keepdims=True)
    sel1 = (r_iota == i1)
    sims2 = jnp.where(sel1, -1e30, sims)
    m2 = jnp.max(sims2, axis=1, keepdims=True)
    i2 = jnp.min(jnp.where(sims2 == m2, r_iota, R), axis=1, keepdims=True)
    sel2 = (r_iota == i2)
    e2 = jnp.exp((m2 - m1) * 10.0)
    a1 = 1.0 / (1.0 + e2)
    a2 = e2 / (1.0 + e2)
    wsel_ref[...] = jnp.where(sel1, a1, 0.0) + jnp.where(sel2, a2, 0.0)


def _decode_body(vals_ref, w1_ref, b1_ref, khw_ref, wsel_ref, w9_ref, b2_ref,
                 out_ref, s0_ref, s1_ref, s2_ref, s3_ref):
    i = pl.program_id(0)
    scrs = [s0_ref, s1_ref, s2_ref, s3_ref]

    @pl.when(i == 0)
    def _zero_pads():
        for sr in scrs:
            _zero_scratch_pads(sr)

    t = _mm(vals_ref[...], w1_ref[...]) + b1_ref[...]
    tb = t.astype(jnp.bfloat16)
    trs = [tb[r * 64:(r + 1) * 64, :] for r in range(R)]
    for s in range(HW // CH):
        ku = khw_ref[pl.ds(s * CH, CH), :]
        gs = [_gelu(_mm(ku, trs[r])) for r in range(R)]
        for b in range(B):
            wb = None
            for r in range(R):
                term = gs[r] * wsel_ref[b, r]
                wb = term if wb is None else wb + term
            scrs[b][pl.ds(PAD + s * CH, CH), :] = wb
    taps = w9_ref[...]
    b2 = b2_ref[...]
    for b in range(B):
        for s in range(HW // CH):
            y = _dwconv_chunk(scrs[b], s * CH, taps) + b2
            out_ref[b, pl.ds(s * CH, CH), :] = y.astype(jnp.bfloat16)


def _fusion1_body(cc_ref, wv_ref, w1a_ref, w1b_ref, b1_ref, fp_ref, st_ref):
    x = cc_ref[0].astype(jnp.bfloat16)
    wv = wv_ref[0]
    f = _mm(x, w1a_ref[...]) + _mm(wv, w1b_ref[...]) + b1_ref[...]
    fp_ref[0] = f.astype(jnp.bfloat16)
    s0 = jnp.sum(f, axis=0, keepdims=True)
    s1 = jnp.sum(f * f, axis=0, keepdims=True)
    upd = jnp.concatenate([s0, s1], axis=1)
    s_idx = pl.program_id(1)

    @pl.when(s_idx == 0)
    def _init():
        st_ref[0] = upd

    @pl.when(s_idx != 0)
    def _acc():
        st_ref[0] = st_ref[0] + upd


def _fusion2_body(fp_ref, st_ref, gsm_ref, gamma_ref, beta_ref, w9_ref, b2_ref,
                  out_ref, scr_ref):
    b = pl.program_id(0)

    @pl.when(b == 0)
    def _zero_pads():
        _zero_scratch_pads(scr_ref)

    st = st_ref[0]
    gsm = gsm_ref[...]
    gs0 = _mm(st[:, :C], gsm)
    gs1 = _mm(st[:, C:], gsm)
    inv_n = 1.0 / (GC * HW)
    mean_g = gs0 * inv_n
    var_g = gs1 * inv_n - mean_g * mean_g
    rstd_g = jax.lax.rsqrt(var_g + 1e-5)
    mean_c = _mm(mean_g, gsm.T)
    rstd_c = _mm(rstd_g, gsm.T)
    sg = rstd_c * gamma_ref[...]
    off = beta_ref[...] - mean_c * sg
    for s in range(HW // CH):
        f = fp_ref[0, pl.ds(s * CH, CH), :].astype(jnp.float32)
        y = _gelu(f * sg + off)
        scr_ref[pl.ds(PAD + s * CH, CH), :] = y
    taps = w9_ref[...]
    b2 = b2_ref[...]
    for s in range(HW // CH):
        y = _dwconv_chunk(scr_ref, s * CH, taps) + b2
        out_ref[0, pl.ds(s * CH, CH), :] = y.astype(jnp.bfloat16)


def _final_body(fm_ref, cc_ref, wv_ref, w3_ref, b3_ref, wga_ref, wgb_ref,
                bg_ref, out_ref):
    x = cc_ref[0]
    xb = x.astype(jnp.bfloat16)
    fo = _mm(fm_ref[0], w3_ref[...]) + b3_ref[...]
    gp = _mm(xb, wga_ref[...]) + _mm(wv_ref[0], wgb_ref[...]) + bg_ref[...]
    gate = jax.nn.sigmoid(gp)
    out_ref[0] = gate * fo + (1.0 - gate) * x


def _full(shape):
    nd = len(shape)
    return pl.BlockSpec(shape, lambda *a: (0,) * nd)


def kernel(current_context, k, Wq, bq, memory_keys, memory_values,
           vd_w1, vd_b1, vd_w2, vd_b2, fn_w1, fn_b1, gn_gamma, gn_beta,
           fn_w2, fn_b2, fn_w3, fn_b3, fg_w, fg_b):
    del k
    f32 = jnp.float32
    bf16 = jnp.bfloat16
    cc = current_context.reshape(B, C, HW)
    cc_t = jnp.transpose(cc, (0, 2, 1))

    U = jax.image.resize(jnp.eye(VR, dtype=f32), (H, VR), method='bilinear')
    khw = jnp.kron(U, U).astype(bf16)
    vals2t = memory_values[0].transpose(0, 2, 3, 1).reshape(R * VR * VR, C)
    keys2 = memory_keys[0, :, :, 0, 0]
    bq2 = bq.reshape(KD, 1)
    w1T = vd_w1.T
    vd_b1r = vd_b1.reshape(1, C)
    vd_w9 = vd_w2.reshape(C, 9).T
    vd_b2r = vd_b2.reshape(1, C)
    w1aT = fn_w1[:, :C].T.astype(bf16)
    w1bT = fn_w1[:, C:].T.astype(bf16)
    fn_b1r = fn_b1.reshape(1, C)
    gsm = jnp.kron(jnp.eye(GROUPS, dtype=f32), jnp.ones((GC, 1), f32))
    gammar = gn_gamma.reshape(1, C)
    betar = gn_beta.reshape(1, C)
    fn_w9 = fn_w2.reshape(C, 9).T
    fn_b2r = fn_b2.reshape(1, C)
    w3b = fn_w3.T.astype(bf16)
    fn_b3c = fn_b3.reshape(1, C)
    wgab = fg_w[:, :C].T.astype(bf16)
    wgbb = fg_w[:, C:].T.astype(bf16)
    fg_bc = fg_b.reshape(1, C)

    qsum = pl.pallas_call(
        _qproj_body,
        grid=(B,),
        in_specs=[
            pl.BlockSpec((1, C, HW), lambda b: (b, 0, 0)),
            pl.BlockSpec((KD, C), lambda b: (0, 0)),
            pl.BlockSpec((KD, 1), lambda b: (0, 0)),
        ],
        out_specs=pl.BlockSpec((1, 1, KD), lambda b: (b, 0, 0)),
        out_shape=jax.ShapeDtypeStruct((B, 1, KD), f32),
    )(cc, Wq, bq2)

    wsel = pl.pallas_call(
        _route_body,
        in_specs=[_full((B, 1, KD)), _full((R, KD))],
        out_specs=_full((B, R)),
        out_shape=jax.ShapeDtypeStruct((B, R), f32),
    )(qsum, keys2)

    CB = 128
    NCB = C // CB
    weighted = pl.pallas_call(
        _decode_body,
        grid=(NCB,),
        in_specs=[
            pl.BlockSpec((R * VR * VR, C), lambda i: (0, 0)),
            pl.BlockSpec((C, CB), lambda i: (0, i)),
            pl.BlockSpec((1, CB), lambda i: (0, i)),
            pl.BlockSpec((HW, VR * VR), lambda i: (0, 0)),
            pl.BlockSpec(memory_space=pltpu.SMEM),
            pl.BlockSpec((9, CB), lambda i: (0, i)),
            pl.BlockSpec((1, CB), lambda i: (0, i)),
        ],
        out_specs=pl.BlockSpec((B, HW, CB), lambda i: (0, 0, i)),
        out_shape=jax.ShapeDtypeStruct((B, HW, C), bf16),
        scratch_shapes=[pltpu.VMEM((SROWS, CB), f32) for _ in range(B)],
    )(vals2t, w1T, vd_b1r, khw, wsel, vd_w9, vd_b2r)

    SC1 = 2048
    NS1 = HW // SC1
    fusion_pre, stats = pl.pallas_call(
        _fusion1_body,
        grid=(B, NS1),
        in_specs=[
            pl.BlockSpec((1, SC1, C), lambda b, s: (b, s, 0)),
            pl.BlockSpec((1, SC1, C), lambda b, s: (b, s, 0)),
            pl.BlockSpec((C, C), lambda b, s: (0, 0)),
            pl.BlockSpec((C, C), lambda b, s: (0, 0)),
            pl.BlockSpec((1, C), lambda b, s: (0, 0)),
        ],
        out_specs=[
            pl.BlockSpec((1, SC1, C), lambda b, s: (b, s, 0)),
            pl.BlockSpec((1, 1, 2 * C), lambda b, s: (b, 0, 0)),
        ],
        out_shape=[
            jax.ShapeDtypeStruct((B, HW, C), bf16),
            jax.ShapeDtypeStruct((B, 1, 2 * C), f32),
        ],
    )(cc_t, weighted, w1aT, w1bT, fn_b1r)

    fusion_mid = pl.pallas_call(
        _fusion2_body,
        grid=(B,),
        in_specs=[
            pl.BlockSpec((1, HW, C), lambda b: (b, 0, 0)),
            pl.BlockSpec((1, 1, 2 * C), lambda b: (b, 0, 0)),
            pl.BlockSpec((C, GROUPS), lambda b: (0, 0)),
            pl.BlockSpec((1, C), lambda b: (0, 0)),
            pl.BlockSpec((1, C), lambda b: (0, 0)),
            pl.BlockSpec((9, C), lambda b: (0, 0)),
            pl.BlockSpec((1, C), lambda b: (0, 0)),
        ],
        out_specs=pl.BlockSpec((1, HW, C), lambda b: (b, 0, 0)),
        out_shape=jax.ShapeDtypeStruct((B, HW, C), bf16),
        scratch_shapes=[pltpu.VMEM((SROWS, C), f32)],
    )(fusion_pre, stats, gsm, gammar, betar, fn_w9, fn_b2r)

    SC2 = 1024
    NS2 = HW // SC2
    out_t = pl.pallas_call(
        _final_body,
        grid=(B, NS2),
        in_specs=[
            pl.BlockSpec((1, SC2, C), lambda b, s: (b, s, 0)),
            pl.BlockSpec((1, SC2, C), lambda b, s: (b, s, 0)),
            pl.BlockSpec((1, SC2, C), lambda b, s: (b, s, 0)),
            pl.BlockSpec((C, C), lambda b, s: (0, 0)),
            pl.BlockSpec((1, C), lambda b, s: (0, 0)),
            pl.BlockSpec((C, C), lambda b, s: (0, 0)),
            pl.BlockSpec((C, C), lambda b, s: (0, 0)),
            pl.BlockSpec((1, C), lambda b, s: (0, 0)),
        ],
        out_specs=pl.BlockSpec((1, SC2, C), lambda b, s: (b, s, 0)),
        out_shape=jax.ShapeDtypeStruct((B, HW, C), f32),
    )(fusion_mid, cc_t, weighted, w3b, fn_b3c, wgab, wgbb, fg_bc)

    return jnp.transpose(out_t, (0, 2, 1)).reshape(B, C, H, W)
